# Initial kernel scaffold; baseline (speedup 1.0000x reference)
import jax
import jax.numpy as jnp
from jax.experimental import pallas as pl


def kernel(pos, batch, params):
    raise NotImplementedError("write your pallas kernel here")


# speedup vs baseline: 6.4849x; 6.4849x over previous
import functools

import jax
import jax.numpy as jnp
from jax import lax
from jax.experimental import pallas as pl
from jax.experimental.pallas import tpu as pltpu
from jax.experimental.pallas import tpu_sc as plsc

_K = 8
_LANE = 128


def _knn_idx(x, xt, bcol, brow, block_rows):
    n, f = x.shape
    grid = n // block_rows

    def body(xr_ref, bc_ref, xt_ref, br_ref, idx_ref):
        xr = xr_ref[...]
        xt_all = xt_ref[...]
        sq_r = jnp.sum(xr * xr, axis=1, keepdims=True)
        sq_c = jnp.sum(xt_all * xt_all, axis=0)[None, :]
        d = sq_r + sq_c - 2.0 * jnp.dot(xr, xt_all, preferred_element_type=jnp.float32)
        d = jnp.where(bc_ref[...] == br_ref[...], d, jnp.inf)
        iota = lax.broadcasted_iota(jnp.int32, d.shape, 1)
        cols = []
        for _ in range(_K):
            mn = jnp.min(d, axis=1, keepdims=True)
            j = jnp.min(jnp.where(d == mn, iota, jnp.int32(n)), axis=1, keepdims=True)
            cols.append(j)
            d = jnp.where(iota == j, jnp.inf, d)
        idx_ref[...] = jnp.concatenate(cols, axis=1)

    return pl.pallas_call(
        body,
        grid=(grid,),
        in_specs=[
            pl.BlockSpec((block_rows, f), lambda i: (i, 0)),
            pl.BlockSpec((block_rows, 1), lambda i: (i, 0)),
            pl.BlockSpec((f, n), lambda i: (0, 0)),
            pl.BlockSpec((1, n), lambda i: (0, 0)),
        ],
        out_specs=pl.BlockSpec((block_rows, _K), lambda i: (i, 0)),
        out_shape=jax.ShapeDtypeStruct((n, _K), jnp.int32),
    )(x, bcol, xt, brow)


def _sc_gather(table, idx_blocks):
    n_blocks, lane = idx_blocks.shape
    d = table.shape[1]
    info = plsc.get_sparse_core_info()
    nc = info.num_cores
    per_w = n_blocks // (nc * info.num_subcores)
    mesh = plsc.VectorSubcoreMesh(core_axis_name="c", subcore_axis_name="s")

    @functools.partial(
        pl.kernel,
        mesh=mesh,
        out_type=jax.ShapeDtypeStruct((n_blocks * lane, d), jnp.float32),
        scratch_types=[
            pltpu.VMEM((per_w, lane), jnp.int32),
            pltpu.VMEM((lane, d), jnp.float32),
            pltpu.SemaphoreType.DMA,
        ],
    )
    def gk(table_hbm, idx_hbm, out_hbm, idx_v, rows_v, sem):
        wid = lax.axis_index("s") * nc + lax.axis_index("c")
        base = wid * per_w
        pltpu.sync_copy(idx_hbm.at[pl.ds(base, per_w)], idx_v)
        for j in range(per_w):
            pltpu.async_copy(table_hbm.at[idx_v.at[j]], rows_v, sem).wait()
            pltpu.sync_copy(rows_v, out_hbm.at[pl.ds((base + j) * lane, lane)])

    return gk(table, idx_blocks)


def _gather_rows(table, idx_flat):
    total = idx_flat.shape[0]
    nw = 32
    blocks = -(-total // _LANE)
    blocks = -(-blocks // nw) * nw
    padded = blocks * _LANE
    idx_p = jnp.concatenate([idx_flat, jnp.zeros((padded - total,), jnp.int32)])
    out = _sc_gather(table, idx_p.reshape(blocks, _LANE))
    return out[:total]


def _edge_lin1(x, xj, w1, b1, bp):
    n, fp = x.shape
    h = w1.shape[1]
    grid = n // bp

    def body(x_ref, xj_ref, w_ref, b_ref, h_ref):
        xi = jnp.broadcast_to(x_ref[...][:, None, :], (bp, _K, fp))
        e = jnp.concatenate([xi, xj_ref[...] - xi], axis=2).reshape(bp * _K, 2 * fp)
        h_ref[...] = jnp.dot(e, w_ref[...], preferred_element_type=jnp.float32) + b_ref[...]

    return pl.pallas_call(
        body,
        grid=(grid,),
        in_specs=[
            pl.BlockSpec((bp, fp), lambda i: (i, 0)),
            pl.BlockSpec((bp, _K, fp), lambda i: (i, 0, 0)),
            pl.BlockSpec(w1.shape, lambda i: (0, 0)),
            pl.BlockSpec((1, h), lambda i: (0, 0)),
        ],
        out_specs=pl.BlockSpec((bp * _K, h), lambda i: (i, 0)),
        out_shape=jax.ShapeDtypeStruct((n * _K, h), jnp.float32),
    )(x, xj, w1, b1.reshape(1, h))


def _edge_bn_lin2_max(h1, mu, denom, gamma, beta, w2, b2, bp):
    nk, h = h1.shape
    n = nk // _K
    fo = w2.shape[1]
    grid = n // bp

    def body(h_ref, mu_ref, dn_ref, g_ref, be_ref, w2_ref, b2_ref, out_ref):
        hm = (h_ref[...] - mu_ref[...]) / dn_ref[...] * g_ref[...] + be_ref[...]
        hm = jnp.maximum(hm, 0.0)
        h2 = jnp.dot(hm, w2_ref[...], preferred_element_type=jnp.float32) + b2_ref[...]
        out_ref[...] = jnp.max(h2.reshape(bp, _K, fo), axis=1)

    return pl.pallas_call(
        body,
        grid=(grid,),
        in_specs=[
            pl.BlockSpec((bp * _K, h), lambda i: (i, 0)),
            pl.BlockSpec((1, h), lambda i: (0, 0)),
            pl.BlockSpec((1, h), lambda i: (0, 0)),
            pl.BlockSpec((1, h), lambda i: (0, 0)),
            pl.BlockSpec((1, h), lambda i: (0, 0)),
            pl.BlockSpec(w2.shape, lambda i: (0, 0)),
            pl.BlockSpec((1, fo), lambda i: (0, 0)),
        ],
        out_specs=pl.BlockSpec((bp, fo), lambda i: (i, 0)),
        out_shape=jax.ShapeDtypeStruct((n, fo), jnp.float32),
    )(h1, mu.reshape(1, h), denom.reshape(1, h), gamma.reshape(1, h),
      beta.reshape(1, h), w2, b2.reshape(1, fo))


def _row_lin1(x, w1, b1, bp):
    n, f = x.shape
    h = w1.shape[1]
    grid = n // bp

    def body(x_ref, w_ref, b_ref, h_ref):
        h_ref[...] = jnp.dot(x_ref[...], w_ref[...],
                             preferred_element_type=jnp.float32) + b_ref[...]

    return pl.pallas_call(
        body,
        grid=(grid,),
        in_specs=[
            pl.BlockSpec((bp, f), lambda i: (i, 0)),
            pl.BlockSpec(w1.shape, lambda i: (0, 0)),
            pl.BlockSpec((1, h), lambda i: (0, 0)),
        ],
        out_specs=pl.BlockSpec((bp, h), lambda i: (i, 0)),
        out_shape=jax.ShapeDtypeStruct((n, h), jnp.float32),
    )(x, w1, b1.reshape(1, h))


def _row_bn_lin2_segmax(h1, bcol, mu, denom, gamma, beta, w2, b2, nseg, bp):
    n, h = h1.shape
    fo = w2.shape[1]
    grid = n // bp

    def body(h_ref, bc_ref, mu_ref, dn_ref, g_ref, be_ref, w2_ref, b2_ref, out_ref):
        i = pl.program_id(0)
        hm = (h_ref[...] - mu_ref[...]) / dn_ref[...] * g_ref[...] + be_ref[...]
        hm = jnp.maximum(hm, 0.0)
        h2 = jnp.dot(hm, w2_ref[...], preferred_element_type=jnp.float32) + b2_ref[...]
        bb = bc_ref[...]
        rows = [jnp.max(jnp.where(bb == s, h2, -jnp.inf), axis=0, keepdims=True)
                for s in range(nseg)]
        cur = jnp.concatenate(rows, axis=0)

        @pl.when(i == 0)
        def _():
            out_ref[...] = jnp.full_like(out_ref[...], -jnp.inf)

        out_ref[...] = jnp.maximum(out_ref[...], cur)

    return pl.pallas_call(
        body,
        grid=(grid,),
        in_specs=[
            pl.BlockSpec((bp, h), lambda i: (i, 0)),
            pl.BlockSpec((bp, 1), lambda i: (i, 0)),
            pl.BlockSpec((1, h), lambda i: (0, 0)),
            pl.BlockSpec((1, h), lambda i: (0, 0)),
            pl.BlockSpec((1, h), lambda i: (0, 0)),
            pl.BlockSpec((1, h), lambda i: (0, 0)),
            pl.BlockSpec(w2.shape, lambda i: (0, 0)),
            pl.BlockSpec((1, fo), lambda i: (0, 0)),
        ],
        out_specs=pl.BlockSpec((nseg, fo), lambda i: (0, 0)),
        out_shape=jax.ShapeDtypeStruct((nseg, fo), jnp.float32),
    )(h1, bcol, mu.reshape(1, h), denom.reshape(1, h), gamma.reshape(1, h),
      beta.reshape(1, h), w2, b2.reshape(1, fo))


def _head(g, w1, b1, w2, b2, bc):
    nseg = g.shape[0]
    hh = w1.shape[1]
    p = w2.shape[1]
    grid = -(-p // bc)

    def body(g_ref, w1_ref, b1_ref, w2_ref, b2_ref, sig_ref, sco_ref):
        hid = jnp.maximum(
            jnp.dot(g_ref[...], w1_ref[...], preferred_element_type=jnp.float32)
            + b1_ref[...], 0.0)
        s = jnp.dot(hid, w2_ref[...], preferred_element_type=jnp.float32) + b2_ref[...]
        sco_ref[...] = s
        sig_ref[...] = jax.nn.sigmoid(s)

    return pl.pallas_call(
        body,
        grid=(grid,),
        in_specs=[
            pl.BlockSpec((nseg, g.shape[1]), lambda i: (0, 0)),
            pl.BlockSpec(w1.shape, lambda i: (0, 0)),
            pl.BlockSpec((1, hh), lambda i: (0, 0)),
            pl.BlockSpec((hh, bc), lambda i: (0, i)),
            pl.BlockSpec((1, bc), lambda i: (0, i)),
        ],
        out_specs=[pl.BlockSpec((nseg, bc), lambda i: (0, i))] * 2,
        out_shape=[jax.ShapeDtypeStruct((nseg, p), jnp.float32)] * 2,
    )(g, w1, b1.reshape(1, hh), w2, b2.reshape(1, p))


def _pad_cols(x, w):
    return jnp.pad(x, ((0, 0), (0, w - x.shape[1]))) if x.shape[1] != w else x


def _edge_conv(xk, bcol, brow, layers, f_true, bp):
    n = xk.shape[0]
    w1, b1 = layers[0]["W"], layers[0]["b"]
    gamma, beta = layers[0]["gamma"], layers[0]["beta"]
    w2, b2 = layers[1]["W"], layers[1]["b"]
    h = w1.shape[1]
    fp = _LANE
    table = _pad_cols(xk, fp)
    w1cat = (jnp.zeros((2 * fp, h), jnp.float32)
             .at[:f_true].set(w1[:f_true])
             .at[fp:fp + f_true].set(w1[f_true:]))
    idx = _knn_idx(xk, xk.T, bcol, brow, 400)
    xj = _gather_rows(table, idx.reshape(n * _K)).reshape(n, _K, fp)
    h1 = _edge_lin1(table, xj, w1cat, b1, bp)
    mu = jnp.mean(h1, axis=0)
    denom = jnp.sqrt(jnp.var(h1, axis=0) + 1e-5)
    return _edge_bn_lin2_max(h1, mu, denom, gamma, beta, w2, b2, bp)


def kernel(pos, batch, params):
    n = pos.shape[0]
    nseg = 4
    batch = batch.astype(jnp.int32)
    bcol = batch.reshape(n, 1)
    brow = batch.reshape(1, n)

    pos_p = _pad_cols(pos, 16)
    x1 = _edge_conv(pos_p, bcol, brow, params["conv1"], 3, 400)
    x2 = _edge_conv(x1, bcol, brow, params["conv2"], 32, 400)
    x3 = _edge_conv(x2, bcol, brow, params["conv3"], 128, 400)

    xcat = jnp.concatenate([x1, x2, x3], axis=1)
    ws = params["shared"]
    hs = _row_lin1(xcat, ws[0]["W"], ws[0]["b"], 400)
    mu = jnp.mean(hs, axis=0)
    denom = jnp.sqrt(jnp.var(hs, axis=0) + 1e-5)
    g = _row_bn_lin2_segmax(hs, bcol, mu, denom, ws[0]["gamma"], ws[0]["beta"],
                            ws[1]["W"], ws[1]["b"], nseg, 400)

    hd = params["head"]
    sig, scores = _head(g, hd["W1"], hd["b1"], hd["W2"], hd["b2"], 8192)
    return (sig, scores)


# speedup vs baseline: 6.9614x; 1.0735x over previous
import functools

import jax
import jax.numpy as jnp
from jax import lax
from jax.experimental import pallas as pl
from jax.experimental.pallas import tpu as pltpu
from jax.experimental.pallas import tpu_sc as plsc

_K = 8
_LANE = 128


def _knn_idx(x, xt, bcol, brow, block_rows):
    n, f = x.shape
    grid = n // block_rows

    def body(xr_ref, bc_ref, xt_ref, br_ref, idx_ref):
        xr = xr_ref[...]
        xt_all = xt_ref[...]
        sq_r = jnp.sum(xr * xr, axis=1, keepdims=True)
        sq_c = jnp.sum(xt_all * xt_all, axis=0)[None, :]
        d = sq_r + sq_c - 2.0 * jnp.dot(xr, xt_all, preferred_element_type=jnp.float32)
        d = jnp.where(bc_ref[...] == br_ref[...], d, jnp.inf)
        iota = lax.broadcasted_iota(jnp.int32, d.shape, 1)
        cols = []
        for _ in range(_K):
            j = jnp.argmin(d, axis=1).astype(jnp.int32)[:, None]
            cols.append(j)
            d = jnp.where(iota == j, jnp.inf, d)
        idx_ref[...] = jnp.concatenate(cols, axis=1)

    return pl.pallas_call(
        body,
        grid=(grid,),
        in_specs=[
            pl.BlockSpec((block_rows, f), lambda i: (i, 0)),
            pl.BlockSpec((block_rows, 1), lambda i: (i, 0)),
            pl.BlockSpec((f, n), lambda i: (0, 0)),
            pl.BlockSpec((1, n), lambda i: (0, 0)),
        ],
        out_specs=pl.BlockSpec((block_rows, _K), lambda i: (i, 0)),
        out_shape=jax.ShapeDtypeStruct((n, _K), jnp.int32),
    )(x, bcol, xt, brow)


def _sc_gather(table, idx_blocks):
    n_blocks, lane = idx_blocks.shape
    d = table.shape[1]
    info = plsc.get_sparse_core_info()
    nc = info.num_cores
    per_w = n_blocks // (nc * info.num_subcores)
    mesh = plsc.VectorSubcoreMesh(core_axis_name="c", subcore_axis_name="s")

    nb = 3

    @functools.partial(
        pl.kernel,
        mesh=mesh,
        out_type=jax.ShapeDtypeStruct((n_blocks * lane, d), jnp.float32),
        scratch_types=[
            pltpu.VMEM((per_w, lane), jnp.int32),
            pltpu.VMEM((nb, lane, d), jnp.float32),
            pltpu.SemaphoreType.DMA(nb),
            pltpu.SemaphoreType.DMA(nb),
        ],
    )
    def gk(table_hbm, idx_hbm, out_hbm, idx_v, rows_v, gsem, wsem):
        wid = lax.axis_index("s") * nc + lax.axis_index("c")
        base = wid * per_w
        pltpu.sync_copy(idx_hbm.at[pl.ds(base, per_w)], idx_v)
        gops = [None] * per_w
        wops = [None] * per_w
        for j in range(min(2, per_w)):
            gops[j] = pltpu.async_copy(table_hbm.at[idx_v.at[j]], rows_v.at[j % nb],
                                       gsem.at[j % nb])
        for j in range(per_w):
            if j + 2 < per_w:
                if j - 1 >= 0:
                    wops[j - 1].wait()
                gops[j + 2] = pltpu.async_copy(table_hbm.at[idx_v.at[j + 2]],
                                               rows_v.at[(j + 2) % nb],
                                               gsem.at[(j + 2) % nb])
            gops[j].wait()
            wops[j] = pltpu.async_copy(rows_v.at[j % nb],
                                       out_hbm.at[pl.ds((base + j) * lane, lane)],
                                       wsem.at[j % nb])
        for j in range(max(0, per_w - 3), per_w):
            if wops[j] is not None:
                wops[j].wait()

    return gk(table, idx_blocks)


def _gather_rows(table, idx_flat):
    total = idx_flat.shape[0]
    nw = 32
    blocks = -(-total // _LANE)
    blocks = -(-blocks // nw) * nw
    padded = blocks * _LANE
    idx_p = jnp.concatenate([idx_flat, jnp.zeros((padded - total,), jnp.int32)])
    out = _sc_gather(table, idx_p.reshape(blocks, _LANE))
    return out[:total]


def _edge_lin1(x, xj, w1, b1, bp):
    n, fp = x.shape
    h = w1.shape[1]
    grid = n // bp

    def body(x_ref, xj_ref, w_ref, b_ref, h_ref):
        xi = jnp.broadcast_to(x_ref[...][:, None, :], (bp, _K, fp))
        e = jnp.concatenate([xi, xj_ref[...] - xi], axis=2).reshape(bp * _K, 2 * fp)
        h_ref[...] = jnp.dot(e, w_ref[...], preferred_element_type=jnp.float32) + b_ref[...]

    return pl.pallas_call(
        body,
        grid=(grid,),
        in_specs=[
            pl.BlockSpec((bp, fp), lambda i: (i, 0)),
            pl.BlockSpec((bp, _K, fp), lambda i: (i, 0, 0)),
            pl.BlockSpec(w1.shape, lambda i: (0, 0)),
            pl.BlockSpec((1, h), lambda i: (0, 0)),
        ],
        out_specs=pl.BlockSpec((bp * _K, h), lambda i: (i, 0)),
        out_shape=jax.ShapeDtypeStruct((n * _K, h), jnp.float32),
    )(x, xj, w1, b1.reshape(1, h))


def _edge_bn_lin2_max(h1, mu, denom, gamma, beta, w2, b2, bp):
    nk, h = h1.shape
    n = nk // _K
    fo = w2.shape[1]
    grid = n // bp

    def body(h_ref, mu_ref, dn_ref, g_ref, be_ref, w2_ref, b2_ref, out_ref):
        hm = (h_ref[...] - mu_ref[...]) / dn_ref[...] * g_ref[...] + be_ref[...]
        hm = jnp.maximum(hm, 0.0)
        h2 = jnp.dot(hm, w2_ref[...], preferred_element_type=jnp.float32) + b2_ref[...]
        out_ref[...] = jnp.max(h2.reshape(bp, _K, fo), axis=1)

    return pl.pallas_call(
        body,
        grid=(grid,),
        in_specs=[
            pl.BlockSpec((bp * _K, h), lambda i: (i, 0)),
            pl.BlockSpec((1, h), lambda i: (0, 0)),
            pl.BlockSpec((1, h), lambda i: (0, 0)),
            pl.BlockSpec((1, h), lambda i: (0, 0)),
            pl.BlockSpec((1, h), lambda i: (0, 0)),
            pl.BlockSpec(w2.shape, lambda i: (0, 0)),
            pl.BlockSpec((1, fo), lambda i: (0, 0)),
        ],
        out_specs=pl.BlockSpec((bp, fo), lambda i: (i, 0)),
        out_shape=jax.ShapeDtypeStruct((n, fo), jnp.float32),
    )(h1, mu.reshape(1, h), denom.reshape(1, h), gamma.reshape(1, h),
      beta.reshape(1, h), w2, b2.reshape(1, fo))


def _row_lin1(x, w1, b1, bp):
    n, f = x.shape
    h = w1.shape[1]
    grid = n // bp

    def body(x_ref, w_ref, b_ref, h_ref):
        h_ref[...] = jnp.dot(x_ref[...], w_ref[...],
                             preferred_element_type=jnp.float32) + b_ref[...]

    return pl.pallas_call(
        body,
        grid=(grid,),
        in_specs=[
            pl.BlockSpec((bp, f), lambda i: (i, 0)),
            pl.BlockSpec(w1.shape, lambda i: (0, 0)),
            pl.BlockSpec((1, h), lambda i: (0, 0)),
        ],
        out_specs=pl.BlockSpec((bp, h), lambda i: (i, 0)),
        out_shape=jax.ShapeDtypeStruct((n, h), jnp.float32),
    )(x, w1, b1.reshape(1, h))


def _row_bn_lin2_segmax(h1, bcol, mu, denom, gamma, beta, w2, b2, nseg, bp):
    n, h = h1.shape
    fo = w2.shape[1]
    grid = n // bp

    def body(h_ref, bc_ref, mu_ref, dn_ref, g_ref, be_ref, w2_ref, b2_ref, out_ref):
        i = pl.program_id(0)
        hm = (h_ref[...] - mu_ref[...]) / dn_ref[...] * g_ref[...] + be_ref[...]
        hm = jnp.maximum(hm, 0.0)
        h2 = jnp.dot(hm, w2_ref[...], preferred_element_type=jnp.float32) + b2_ref[...]
        bb = bc_ref[...]
        rows = [jnp.max(jnp.where(bb == s, h2, -jnp.inf), axis=0, keepdims=True)
                for s in range(nseg)]
        cur = jnp.concatenate(rows, axis=0)

        @pl.when(i == 0)
        def _():
            out_ref[...] = jnp.full_like(out_ref[...], -jnp.inf)

        out_ref[...] = jnp.maximum(out_ref[...], cur)

    return pl.pallas_call(
        body,
        grid=(grid,),
        in_specs=[
            pl.BlockSpec((bp, h), lambda i: (i, 0)),
            pl.BlockSpec((bp, 1), lambda i: (i, 0)),
            pl.BlockSpec((1, h), lambda i: (0, 0)),
            pl.BlockSpec((1, h), lambda i: (0, 0)),
            pl.BlockSpec((1, h), lambda i: (0, 0)),
            pl.BlockSpec((1, h), lambda i: (0, 0)),
            pl.BlockSpec(w2.shape, lambda i: (0, 0)),
            pl.BlockSpec((1, fo), lambda i: (0, 0)),
        ],
        out_specs=pl.BlockSpec((nseg, fo), lambda i: (0, 0)),
        out_shape=jax.ShapeDtypeStruct((nseg, fo), jnp.float32),
    )(h1, bcol, mu.reshape(1, h), denom.reshape(1, h), gamma.reshape(1, h),
      beta.reshape(1, h), w2, b2.reshape(1, fo))


def _head(g, w1, b1, w2, b2, bc):
    nseg = g.shape[0]
    hh = w1.shape[1]
    p = w2.shape[1]
    grid = -(-p // bc)

    def body(g_ref, w1_ref, b1_ref, w2_ref, b2_ref, sig_ref, sco_ref):
        hid = jnp.maximum(
            jnp.dot(g_ref[...], w1_ref[...], preferred_element_type=jnp.float32)
            + b1_ref[...], 0.0)
        s = jnp.dot(hid, w2_ref[...], preferred_element_type=jnp.float32) + b2_ref[...]
        sco_ref[...] = s
        sig_ref[...] = jax.nn.sigmoid(s)

    return pl.pallas_call(
        body,
        grid=(grid,),
        in_specs=[
            pl.BlockSpec((nseg, g.shape[1]), lambda i: (0, 0)),
            pl.BlockSpec(w1.shape, lambda i: (0, 0)),
            pl.BlockSpec((1, hh), lambda i: (0, 0)),
            pl.BlockSpec((hh, bc), lambda i: (0, i)),
            pl.BlockSpec((1, bc), lambda i: (0, i)),
        ],
        out_specs=[pl.BlockSpec((nseg, bc), lambda i: (0, i))] * 2,
        out_shape=[jax.ShapeDtypeStruct((nseg, p), jnp.float32)] * 2,
    )(g, w1, b1.reshape(1, hh), w2, b2.reshape(1, p))


def _pad_cols(x, w):
    return jnp.pad(x, ((0, 0), (0, w - x.shape[1]))) if x.shape[1] != w else x


def _edge_conv(xk, bcol, brow, layers, f_true, bp):
    n = xk.shape[0]
    w1, b1 = layers[0]["W"], layers[0]["b"]
    gamma, beta = layers[0]["gamma"], layers[0]["beta"]
    w2, b2 = layers[1]["W"], layers[1]["b"]
    h = w1.shape[1]
    fp = _LANE
    table = _pad_cols(xk, fp)
    w1cat = (jnp.zeros((2 * fp, h), jnp.float32)
             .at[:f_true].set(w1[:f_true])
             .at[fp:fp + f_true].set(w1[f_true:]))
    idx = _knn_idx(xk, xk.T, bcol, brow, 400)
    xj = _gather_rows(table, idx.reshape(n * _K)).reshape(n, _K, fp)
    h1 = _edge_lin1(table, xj, w1cat, b1, bp)
    mu = jnp.mean(h1, axis=0)
    denom = jnp.sqrt(jnp.var(h1, axis=0) + 1e-5)
    return _edge_bn_lin2_max(h1, mu, denom, gamma, beta, w2, b2, bp)


def kernel(pos, batch, params):
    n = pos.shape[0]
    nseg = 4
    batch = batch.astype(jnp.int32)
    bcol = batch.reshape(n, 1)
    brow = batch.reshape(1, n)

    pos_p = _pad_cols(pos, 16)
    x1 = _edge_conv(pos_p, bcol, brow, params["conv1"], 3, 400)
    x2 = _edge_conv(x1, bcol, brow, params["conv2"], 32, 400)
    x3 = _edge_conv(x2, bcol, brow, params["conv3"], 128, 400)

    xcat = jnp.concatenate([x1, x2, x3], axis=1)
    ws = params["shared"]
    hs = _row_lin1(xcat, ws[0]["W"], ws[0]["b"], 400)
    mu = jnp.mean(hs, axis=0)
    denom = jnp.sqrt(jnp.var(hs, axis=0) + 1e-5)
    g = _row_bn_lin2_segmax(hs, bcol, mu, denom, ws[0]["gamma"], ws[0]["beta"],
                            ws[1]["W"], ws[1]["b"], nseg, 400)

    hd = params["head"]
    sig, scores = _head(g, hd["W1"], hd["b1"], hd["W2"], hd["b2"], 8192)
    return (sig, scores)


# speedup vs baseline: 7.7161x; 1.1084x over previous
import functools

import jax
import jax.numpy as jnp
from jax import lax
from jax.experimental import pallas as pl
from jax.experimental.pallas import tpu as pltpu
from jax.experimental.pallas import tpu_sc as plsc
from jax._src.pallas import core as pl_core

_K = 8
_LANE = 128


def _knn_idx(x, xt, bcol, brow, block_rows):
    n, f = x.shape
    grid = n // block_rows

    def body(xr_ref, bc_ref, xt_ref, br_ref, idx_ref):
        xr = xr_ref[...]
        xt_all = xt_ref[...]
        sq_r = jnp.sum(xr * xr, axis=1, keepdims=True)
        sq_c = jnp.sum(xt_all * xt_all, axis=0)[None, :]
        d = sq_r + sq_c - 2.0 * jnp.dot(xr, xt_all, preferred_element_type=jnp.float32)
        d = jnp.where(bc_ref[...] == br_ref[...], d, jnp.inf)
        iota = lax.broadcasted_iota(jnp.int32, d.shape, 1)
        cols = []
        for _ in range(_K):
            j = jnp.argmin(d, axis=1).astype(jnp.int32)[:, None]
            cols.append(j)
            d = jnp.where(iota == j, jnp.inf, d)
        idx_ref[...] = jnp.concatenate(cols, axis=1)

    return pl.pallas_call(
        body,
        grid=(grid,),
        in_specs=[
            pl.BlockSpec((block_rows, f), lambda i: (i, 0)),
            pl.BlockSpec((block_rows, 1), lambda i: (i, 0)),
            pl.BlockSpec((f, n), lambda i: (0, 0)),
            pl.BlockSpec((1, n), lambda i: (0, 0)),
        ],
        out_specs=pl.BlockSpec((block_rows, _K), lambda i: (i, 0)),
        out_shape=jax.ShapeDtypeStruct((n, _K), jnp.int32),
    )(x, bcol, xt, brow)


_WIN = 2560


def _knn_idx_win(x, xt_pad, bcol, brow_pad, cstart, block_rows):
    n, f = x.shape
    grid = n // block_rows

    def body(cs_ref, xr_ref, bc_ref, xt_ref, br_ref, idx_ref):
        i = pl.program_id(0)
        base = cs_ref[i]
        xr = xr_ref[...]
        xt_w = xt_ref[...]
        sq_r = jnp.sum(xr * xr, axis=1, keepdims=True)
        sq_c = jnp.sum(xt_w * xt_w, axis=0)[None, :]
        d = sq_r + sq_c - 2.0 * jnp.dot(xr, xt_w, preferred_element_type=jnp.float32)
        d = jnp.where(bc_ref[...] == br_ref[...], d, jnp.inf)
        iota = lax.broadcasted_iota(jnp.int32, d.shape, 1)
        cols = []
        for _ in range(_K):
            j = jnp.argmin(d, axis=1).astype(jnp.int32)[:, None]
            cols.append(j + base)
            d = jnp.where(iota == j, jnp.inf, d)
        idx_ref[...] = jnp.concatenate(cols, axis=1)

    grid_spec = pltpu.PrefetchScalarGridSpec(
        num_scalar_prefetch=1,
        grid=(grid,),
        in_specs=[
            pl.BlockSpec((block_rows, f), lambda i, cs: (i, 0)),
            pl.BlockSpec((block_rows, 1), lambda i, cs: (i, 0)),
            pl.BlockSpec((pl_core.Element(f), pl_core.Element(_WIN)),
                         lambda i, cs: (0, pl.multiple_of(cs[i], 128))),
            pl.BlockSpec((pl_core.Element(1), pl_core.Element(_WIN)),
                         lambda i, cs: (0, pl.multiple_of(cs[i], 128))),
        ],
        out_specs=pl.BlockSpec((block_rows, _K), lambda i, cs: (i, 0)),
    )
    return pl.pallas_call(
        body,
        grid_spec=grid_spec,
        out_shape=jax.ShapeDtypeStruct((n, _K), jnp.int32),
    )(cstart, x, bcol, xt_pad, brow_pad)


def _knn_dispatch(xk, bcol, brow, meta, block_rows):
    cstart, ovr_any, ovr_rows, brow_pad = meta
    n, f = xk.shape
    npad = brow_pad.shape[1]
    xt = xk.T
    xt_pad = jnp.pad(xt, ((0, 0), (0, npad - n)))
    idx_a = _knn_idx_win(xk, xt_pad, bcol, brow_pad, cstart, block_rows)
    return lax.cond(
        ovr_any,
        lambda: jnp.where(ovr_rows, _knn_idx(xk, xt, bcol, brow, block_rows), idx_a),
        lambda: idx_a,
    )


def _sc_gather(table, idx_blocks):
    n_blocks, lane = idx_blocks.shape
    d = table.shape[1]
    info = plsc.get_sparse_core_info()
    nc = info.num_cores
    per_w = n_blocks // (nc * info.num_subcores)
    mesh = plsc.VectorSubcoreMesh(core_axis_name="c", subcore_axis_name="s")

    nb = 3

    @functools.partial(
        pl.kernel,
        mesh=mesh,
        out_type=jax.ShapeDtypeStruct((n_blocks * lane, d), jnp.float32),
        scratch_types=[
            pltpu.VMEM((per_w, lane), jnp.int32),
            pltpu.VMEM((nb, lane, d), jnp.float32),
            pltpu.SemaphoreType.DMA(nb),
            pltpu.SemaphoreType.DMA(nb),
        ],
    )
    def gk(table_hbm, idx_hbm, out_hbm, idx_v, rows_v, gsem, wsem):
        wid = lax.axis_index("s") * nc + lax.axis_index("c")
        base = wid * per_w
        pltpu.sync_copy(idx_hbm.at[pl.ds(base, per_w)], idx_v)
        gops = [None] * per_w
        wops = [None] * per_w
        for j in range(min(2, per_w)):
            gops[j] = pltpu.async_copy(table_hbm.at[idx_v.at[j]], rows_v.at[j % nb],
                                       gsem.at[j % nb])
        for j in range(per_w):
            if j + 2 < per_w:
                if j - 1 >= 0:
                    wops[j - 1].wait()
                gops[j + 2] = pltpu.async_copy(table_hbm.at[idx_v.at[j + 2]],
                                               rows_v.at[(j + 2) % nb],
                                               gsem.at[(j + 2) % nb])
            gops[j].wait()
            wops[j] = pltpu.async_copy(rows_v.at[j % nb],
                                       out_hbm.at[pl.ds((base + j) * lane, lane)],
                                       wsem.at[j % nb])
        for j in range(max(0, per_w - 3), per_w):
            if wops[j] is not None:
                wops[j].wait()

    return gk(table, idx_blocks)


def _gather_rows(table, idx_flat):
    total = idx_flat.shape[0]
    nw = 32
    blocks = -(-total // _LANE)
    blocks = -(-blocks // nw) * nw
    padded = blocks * _LANE
    idx_p = jnp.concatenate([idx_flat, jnp.zeros((padded - total,), jnp.int32)])
    out = _sc_gather(table, idx_p.reshape(blocks, _LANE))
    return out[:total]


def _edge_lin1(x, xj, w1, b1, bp):
    n, fp = x.shape
    h = w1.shape[1]
    grid = n // bp

    def body(x_ref, xj_ref, w_ref, b_ref, h_ref):
        xi = jnp.broadcast_to(x_ref[...][:, None, :], (bp, _K, fp))
        e = jnp.concatenate([xi, xj_ref[...] - xi], axis=2).reshape(bp * _K, 2 * fp)
        h_ref[...] = jnp.dot(e, w_ref[...], preferred_element_type=jnp.float32) + b_ref[...]

    return pl.pallas_call(
        body,
        grid=(grid,),
        in_specs=[
            pl.BlockSpec((bp, fp), lambda i: (i, 0)),
            pl.BlockSpec((bp, _K, fp), lambda i: (i, 0, 0)),
            pl.BlockSpec(w1.shape, lambda i: (0, 0)),
            pl.BlockSpec((1, h), lambda i: (0, 0)),
        ],
        out_specs=pl.BlockSpec((bp * _K, h), lambda i: (i, 0)),
        out_shape=jax.ShapeDtypeStruct((n * _K, h), jnp.float32),
    )(x, xj, w1, b1.reshape(1, h))


def _edge_bn_lin2_max(h1, mu, denom, gamma, beta, w2, b2, bp):
    nk, h = h1.shape
    n = nk // _K
    fo = w2.shape[1]
    grid = n // bp

    def body(h_ref, mu_ref, dn_ref, g_ref, be_ref, w2_ref, b2_ref, out_ref):
        hm = (h_ref[...] - mu_ref[...]) / dn_ref[...] * g_ref[...] + be_ref[...]
        hm = jnp.maximum(hm, 0.0)
        h2 = jnp.dot(hm, w2_ref[...], preferred_element_type=jnp.float32) + b2_ref[...]
        out_ref[...] = jnp.max(h2.reshape(bp, _K, fo), axis=1)

    return pl.pallas_call(
        body,
        grid=(grid,),
        in_specs=[
            pl.BlockSpec((bp * _K, h), lambda i: (i, 0)),
            pl.BlockSpec((1, h), lambda i: (0, 0)),
            pl.BlockSpec((1, h), lambda i: (0, 0)),
            pl.BlockSpec((1, h), lambda i: (0, 0)),
            pl.BlockSpec((1, h), lambda i: (0, 0)),
            pl.BlockSpec(w2.shape, lambda i: (0, 0)),
            pl.BlockSpec((1, fo), lambda i: (0, 0)),
        ],
        out_specs=pl.BlockSpec((bp, fo), lambda i: (i, 0)),
        out_shape=jax.ShapeDtypeStruct((n, fo), jnp.float32),
    )(h1, mu.reshape(1, h), denom.reshape(1, h), gamma.reshape(1, h),
      beta.reshape(1, h), w2, b2.reshape(1, fo))


def _row_lin1(x, w1, b1, bp):
    n, f = x.shape
    h = w1.shape[1]
    grid = n // bp

    def body(x_ref, w_ref, b_ref, h_ref):
        h_ref[...] = jnp.dot(x_ref[...], w_ref[...],
                             preferred_element_type=jnp.float32) + b_ref[...]

    return pl.pallas_call(
        body,
        grid=(grid,),
        in_specs=[
            pl.BlockSpec((bp, f), lambda i: (i, 0)),
            pl.BlockSpec(w1.shape, lambda i: (0, 0)),
            pl.BlockSpec((1, h), lambda i: (0, 0)),
        ],
        out_specs=pl.BlockSpec((bp, h), lambda i: (i, 0)),
        out_shape=jax.ShapeDtypeStruct((n, h), jnp.float32),
    )(x, w1, b1.reshape(1, h))


def _row_bn_lin2_segmax(h1, bcol, mu, denom, gamma, beta, w2, b2, nseg, bp):
    n, h = h1.shape
    fo = w2.shape[1]
    grid = n // bp

    def body(h_ref, bc_ref, mu_ref, dn_ref, g_ref, be_ref, w2_ref, b2_ref, out_ref):
        i = pl.program_id(0)
        hm = (h_ref[...] - mu_ref[...]) / dn_ref[...] * g_ref[...] + be_ref[...]
        hm = jnp.maximum(hm, 0.0)
        h2 = jnp.dot(hm, w2_ref[...], preferred_element_type=jnp.float32) + b2_ref[...]
        bb = bc_ref[...]
        rows = [jnp.max(jnp.where(bb == s, h2, -jnp.inf), axis=0, keepdims=True)
                for s in range(nseg)]
        cur = jnp.concatenate(rows, axis=0)

        @pl.when(i == 0)
        def _():
            out_ref[...] = jnp.full_like(out_ref[...], -jnp.inf)

        out_ref[...] = jnp.maximum(out_ref[...], cur)

    return pl.pallas_call(
        body,
        grid=(grid,),
        in_specs=[
            pl.BlockSpec((bp, h), lambda i: (i, 0)),
            pl.BlockSpec((bp, 1), lambda i: (i, 0)),
            pl.BlockSpec((1, h), lambda i: (0, 0)),
            pl.BlockSpec((1, h), lambda i: (0, 0)),
            pl.BlockSpec((1, h), lambda i: (0, 0)),
            pl.BlockSpec((1, h), lambda i: (0, 0)),
            pl.BlockSpec(w2.shape, lambda i: (0, 0)),
            pl.BlockSpec((1, fo), lambda i: (0, 0)),
        ],
        out_specs=pl.BlockSpec((nseg, fo), lambda i: (0, 0)),
        out_shape=jax.ShapeDtypeStruct((nseg, fo), jnp.float32),
    )(h1, bcol, mu.reshape(1, h), denom.reshape(1, h), gamma.reshape(1, h),
      beta.reshape(1, h), w2, b2.reshape(1, fo))


def _head(g, w1, b1, w2, b2, bc):
    nseg = g.shape[0]
    hh = w1.shape[1]
    p = w2.shape[1]
    grid = -(-p // bc)

    def body(g_ref, w1_ref, b1_ref, w2_ref, b2_ref, sig_ref, sco_ref):
        hid = jnp.maximum(
            jnp.dot(g_ref[...], w1_ref[...], preferred_element_type=jnp.float32)
            + b1_ref[...], 0.0)
        s = jnp.dot(hid, w2_ref[...], preferred_element_type=jnp.float32) + b2_ref[...]
        sco_ref[...] = s
        sig_ref[...] = jax.nn.sigmoid(s)

    return pl.pallas_call(
        body,
        grid=(grid,),
        in_specs=[
            pl.BlockSpec((nseg, g.shape[1]), lambda i: (0, 0)),
            pl.BlockSpec(w1.shape, lambda i: (0, 0)),
            pl.BlockSpec((1, hh), lambda i: (0, 0)),
            pl.BlockSpec((hh, bc), lambda i: (0, i)),
            pl.BlockSpec((1, bc), lambda i: (0, i)),
        ],
        out_specs=[pl.BlockSpec((nseg, bc), lambda i: (0, i))] * 2,
        out_shape=[jax.ShapeDtypeStruct((nseg, p), jnp.float32)] * 2,
    )(g, w1, b1.reshape(1, hh), w2, b2.reshape(1, p))


def _pad_cols(x, w):
    return jnp.pad(x, ((0, 0), (0, w - x.shape[1]))) if x.shape[1] != w else x


def _edge_conv(xk, bcol, brow, meta, layers, f_true, bp):
    n = xk.shape[0]
    w1, b1 = layers[0]["W"], layers[0]["b"]
    gamma, beta = layers[0]["gamma"], layers[0]["beta"]
    w2, b2 = layers[1]["W"], layers[1]["b"]
    h = w1.shape[1]
    fp = _LANE
    table = _pad_cols(xk, fp)
    w1cat = (jnp.zeros((2 * fp, h), jnp.float32)
             .at[:f_true].set(w1[:f_true])
             .at[fp:fp + f_true].set(w1[f_true:]))
    idx = _knn_dispatch(xk, bcol, brow, meta, 400)
    xj = _gather_rows(table, idx.reshape(n * _K)).reshape(n, _K, fp)
    h1 = _edge_lin1(table, xj, w1cat, b1, bp)
    mu = jnp.mean(h1, axis=0)
    denom = jnp.sqrt(jnp.var(h1, axis=0) + 1e-5)
    return _edge_bn_lin2_max(h1, mu, denom, gamma, beta, w2, b2, bp)


def kernel(pos, batch, params):
    n = pos.shape[0]
    nseg = 4
    batch = batch.astype(jnp.int32)
    bcol = batch.reshape(n, 1)
    brow = batch.reshape(1, n)

    npad = max(((n + 127) // 128) * 128, _WIN)
    block_rows = 400
    segs = jnp.arange(nseg, dtype=jnp.int32)
    seg_lo = jnp.searchsorted(batch, segs).astype(jnp.int32)
    seg_hi = jnp.searchsorted(batch, segs, side="right").astype(jnp.int32)
    b_first = batch[::block_rows]
    b_last = batch[block_rows - 1::block_rows]
    lo = (seg_lo[b_first] // 128) * 128
    hi = seg_hi[b_last]
    cstart = jnp.minimum(lo, npad - _WIN)
    ovr = (hi - cstart) > _WIN
    ovr_rows = jnp.repeat(ovr, block_rows)[:, None]
    brow_pad = jnp.full((1, npad), -1, jnp.int32).at[0, :n].set(batch)
    meta = (cstart, jnp.any(ovr), ovr_rows, brow_pad)

    pos_p = _pad_cols(pos, 16)
    x1 = _edge_conv(pos_p, bcol, brow, meta, params["conv1"], 3, 400)
    x2 = _edge_conv(x1, bcol, brow, meta, params["conv2"], 32, 400)
    x3 = _edge_conv(x2, bcol, brow, meta, params["conv3"], 128, 400)

    xcat = jnp.concatenate([x1, x2, x3], axis=1)
    ws = params["shared"]
    hs = _row_lin1(xcat, ws[0]["W"], ws[0]["b"], 400)
    mu = jnp.mean(hs, axis=0)
    denom = jnp.sqrt(jnp.var(hs, axis=0) + 1e-5)
    g = _row_bn_lin2_segmax(hs, bcol, mu, denom, ws[0]["gamma"], ws[0]["beta"],
                            ws[1]["W"], ws[1]["b"], nseg, 400)

    hd = params["head"]
    sig, scores = _head(g, hd["W1"], hd["b1"], hd["W2"], hd["b2"], 8192)
    return (sig, scores)
